# Initial kernel scaffold; baseline (speedup 1.0000x reference)
import jax
import jax.numpy as jnp
from jax.experimental import pallas as pl


def kernel(node_features, node_attrs, edge_attrs, edge_embedding, edge_index, W_lin1, W_fc1, W_fc2, W_lin2, W_sc):
    raise NotImplementedError("write your pallas kernel here")


# speedup vs baseline: 1.6180x; 1.6180x over previous
import functools
import math

import jax
import jax.numpy as jnp
import numpy as np
from jax import lax
from jax.experimental import pallas as pl
from jax.experimental.pallas import tpu as pltpu
from jax.experimental.pallas import tpu_sc as plsc

N = 10000
E = 320000
D = 128
A = 16
B = 8
H = 64

_z = np.linspace(-10.0, 10.0, 200001)
_pdf = np.exp(-0.5 * _z ** 2) / np.sqrt(2.0 * np.pi)
_silu_np = _z / (1.0 + np.exp(-_z))
_NSILU_CST = float(1.0 / np.sqrt(np.trapz(_silu_np ** 2 * _pdf, _z)))

_HI = jax.lax.Precision.HIGHEST


def _nsilu(v):
    return jax.nn.silu(v) * _NSILU_CST


_BE = 3200


def _edge_mlp_body(emb_ref, ea_ref, wfc1_ref, wfc2_ref, out_ref):
    h = jnp.dot(emb_ref[...], wfc1_ref[...], precision=_HI,
                preferred_element_type=jnp.float32) * (1.0 / math.sqrt(B))
    h = _nsilu(h)
    w = jnp.dot(h, wfc2_ref[...], precision=_HI,
                preferred_element_type=jnp.float32) * (1.0 / math.sqrt(H))
    out_ref[...] = w * ea_ref[...]


def _edge_mlp(emb, ea, wfc1, wfc2):
    grid = (E // _BE,)
    return pl.pallas_call(
        _edge_mlp_body,
        grid=grid,
        in_specs=[
            pl.BlockSpec((_BE, B), lambda i: (i, 0)),
            pl.BlockSpec((_BE, 1), lambda i: (i, 0)),
            pl.BlockSpec((B, H), lambda i: (0, 0)),
            pl.BlockSpec((H, D), lambda i: (0, 0)),
        ],
        out_specs=pl.BlockSpec((_BE, D), lambda i: (i, 0)),
        out_shape=jax.ShapeDtypeStruct((E, D), jnp.float32),
    )(emb, ea, wfc1, wfc2)


_BN1 = 2000


def _node_lin_body(x0_ref, w_ref, out_ref):
    out_ref[...] = jnp.dot(x0_ref[...], w_ref[...], precision=_HI,
                           preferred_element_type=jnp.float32) * (1.0 / math.sqrt(D))


def _node_linear(x0, wlin1):
    grid = (N // _BN1,)
    return pl.pallas_call(
        _node_lin_body,
        grid=grid,
        in_specs=[
            pl.BlockSpec((_BN1, D), lambda i: (i, 0)),
            pl.BlockSpec((D, D), lambda i: (0, 0)),
        ],
        out_specs=pl.BlockSpec((_BN1, D), lambda i: (i, 0)),
        out_shape=jax.ShapeDtypeStruct((N, D), jnp.float32),
    )(x0, wlin1)


_NC = 2
_NS = 16
_NW = _NC * _NS
_CH = 128
_EPW = E // _NW
_NFULL = _EPW // _CH
_REM = _EPW - _NFULL * _CH
_RPT = 624
_RTAIL = N - _RPT * _NS

@functools.lru_cache(maxsize=1)
def _make_sc_kernel():
    mesh = plsc.VectorSubcoreMesh(core_axis_name="c", subcore_axis_name="s",
                                  num_cores=_NC, num_subcores=_NS)
    return pl.kernel(
        _sc_body,
        out_type=jax.ShapeDtypeStruct((_NC, N, D), jnp.float32),
        mesh=mesh,
        scratch_types=[
            pltpu.VMEM_SHARED((N, D), jnp.float32),
            pltpu.VMEM((_CH,), jnp.int32),
            pltpu.VMEM((_CH,), jnp.int32),
            pltpu.VMEM((_CH, D), jnp.float32),
            pltpu.VMEM((_CH, D), jnp.float32),
            pltpu.VMEM((_REM,), jnp.int32),
            pltpu.VMEM((_REM,), jnp.int32),
            pltpu.VMEM((_REM, D), jnp.float32),
            pltpu.VMEM((_REM, D), jnp.float32),
            pltpu.SemaphoreType.DMA,
        ],
    )


def _sc_body(x_hbm, wq_hbm, ei_hbm, ej_hbm, zero_hbm, out_hbm,
             acc, ei_v, ej_v, w_v, xg_v, ei_r, ej_r, w_r, xg_r, sem):
    cid = lax.axis_index("c")
    sid = lax.axis_index("s")
    pltpu.sync_copy(zero_hbm.at[pl.ds(sid * _RPT, _RPT)],
                    acc.at[pl.ds(sid * _RPT, _RPT)])

    @pl.when(sid == 0)
    def _zero_tail():
        pltpu.sync_copy(zero_hbm.at[pl.ds(_RPT * _NS, _RTAIL)],
                        acc.at[pl.ds(_RPT * _NS, _RTAIL)])
    plsc.subcore_barrier()

    wid = cid * _NS + sid
    ebase = wid * _EPW

    def chunk(k, carry):
        base = pl.multiple_of(ebase + k * _CH, 8)
        pltpu.sync_copy(ei_hbm.at[pl.ds(base, _CH)], ei_v)
        pltpu.sync_copy(ej_hbm.at[pl.ds(base, _CH)], ej_v)
        pltpu.sync_copy(wq_hbm.at[pl.ds(base, _CH)], w_v)
        pltpu.async_copy(x_hbm.at[ei_v], xg_v, sem).wait()

        def row(i, c2):
            for j in range(D // 16):
                sl = pl.ds(j * 16, 16)
                xg_v[i, sl] = xg_v[i, sl] * w_v[i, sl]
            return c2
        lax.fori_loop(0, _CH, row, 0)
        pltpu.sync_copy(xg_v, acc.at[ej_v], add=True)
        return carry

    lax.fori_loop(0, _NFULL, chunk, 0)

    rbase = pl.multiple_of(ebase + _NFULL * _CH, 8)
    pltpu.sync_copy(ei_hbm.at[pl.ds(rbase, _REM)], ei_r)
    pltpu.sync_copy(ej_hbm.at[pl.ds(rbase, _REM)], ej_r)
    pltpu.sync_copy(wq_hbm.at[pl.ds(rbase, _REM)], w_r)
    pltpu.async_copy(x_hbm.at[ei_r], xg_r, sem).wait()

    def rrow(i, c2):
        for j in range(D // 16):
            sl = pl.ds(j * 16, 16)
            xg_r[i, sl] = xg_r[i, sl] * w_r[i, sl]
        return c2
    lax.fori_loop(0, _REM, rrow, 0)
    pltpu.sync_copy(xg_r, acc.at[ej_r], add=True)

    plsc.subcore_barrier()
    pltpu.sync_copy(acc.at[pl.ds(sid * _RPT, _RPT)],
                    out_hbm.at[cid, pl.ds(sid * _RPT, _RPT)])

    @pl.when(sid == 0)
    def _dump_tail():
        pltpu.sync_copy(acc.at[pl.ds(_RPT * _NS, _RTAIL)],
                        out_hbm.at[cid, pl.ds(_RPT * _NS, _RTAIL)])


_BN3 = 400


def _final_body(agg2_ref, x0_ref, attrs_ref, wlin2_ref, wsc2d_ref, out_ref):
    agg = agg2_ref[0] + agg2_ref[1]
    z = jnp.dot(agg, wlin2_ref[...], precision=_HI,
                preferred_element_type=jnp.float32) * (1.0 / math.sqrt(D))
    x0 = x0_ref[...]
    t = jnp.dot(x0, wsc2d_ref[...], precision=_HI,
                preferred_element_type=jnp.float32)
    attrs = attrs_ref[...]
    sc = t[:, 0:D] * attrs[:, 0:1]
    for v in range(1, A):
        sc = sc + t[:, v * D:(v + 1) * D] * attrs[:, v:v + 1]
    sc = sc * (1.0 / math.sqrt(float(D * A)))
    out_ref[...] = _nsilu(z + sc) + x0


def _final(agg2, x0, attrs, wlin2, wsc2d):
    grid = (N // _BN3,)
    return pl.pallas_call(
        _final_body,
        grid=grid,
        in_specs=[
            pl.BlockSpec((_NC, _BN3, D), lambda i: (0, i, 0)),
            pl.BlockSpec((_BN3, D), lambda i: (i, 0)),
            pl.BlockSpec((_BN3, A), lambda i: (i, 0)),
            pl.BlockSpec((D, D), lambda i: (0, 0)),
            pl.BlockSpec((D, A * D), lambda i: (0, 0)),
        ],
        out_specs=pl.BlockSpec((_BN3, D), lambda i: (i, 0)),
        out_shape=jax.ShapeDtypeStruct((N, D), jnp.float32),
    )(agg2, x0, attrs, wlin2, wsc2d)


def kernel(node_features, node_attrs, edge_attrs, edge_embedding, edge_index,
           W_lin1, W_fc1, W_fc2, W_lin2, W_sc):
    ei = edge_index[0]
    ej = edge_index[1]
    x = _node_linear(node_features, W_lin1)
    wq = _edge_mlp(edge_embedding, edge_attrs, W_fc1, W_fc2)
    zeros = jnp.zeros((N, D), jnp.float32)
    agg2 = _make_sc_kernel()(x, wq, ei, ej, zeros)
    return _final(agg2, node_features, node_attrs, W_lin2, W_sc.reshape(D, A * D))


# speedup vs baseline: 2.3880x; 1.4759x over previous
import functools
import math

import jax
import jax.numpy as jnp
import numpy as np
from jax import lax
from jax.experimental import pallas as pl
from jax.experimental.pallas import tpu as pltpu
from jax.experimental.pallas import tpu_sc as plsc

N = 10000
E = 320000
D = 128
A = 16
B = 8
H = 64

_z = np.linspace(-10.0, 10.0, 200001)
_pdf = np.exp(-0.5 * _z ** 2) / np.sqrt(2.0 * np.pi)
_silu_np = _z / (1.0 + np.exp(-_z))
_NSILU_CST = float(1.0 / np.sqrt(np.trapz(_silu_np ** 2 * _pdf, _z)))

_HI = jax.lax.Precision.DEFAULT


def _nsilu(v):
    return jax.nn.silu(v) * _NSILU_CST


_BE = 3200


def _edge_mlp_body(emb_ref, ea_ref, wfc1_ref, wfc2_ref, out_ref):
    h = jnp.dot(emb_ref[...], wfc1_ref[...], precision=_HI,
                preferred_element_type=jnp.float32) * (1.0 / math.sqrt(B))
    h = _nsilu(h)
    w = jnp.dot(h, wfc2_ref[...], precision=_HI,
                preferred_element_type=jnp.float32) * (1.0 / math.sqrt(H))
    out_ref[...] = w * ea_ref[...]


def _edge_mlp(emb, ea, wfc1, wfc2):
    grid = (E // _BE,)
    return pl.pallas_call(
        _edge_mlp_body,
        grid=grid,
        in_specs=[
            pl.BlockSpec((_BE, B), lambda i: (i, 0)),
            pl.BlockSpec((_BE, 1), lambda i: (i, 0)),
            pl.BlockSpec((B, H), lambda i: (0, 0)),
            pl.BlockSpec((H, D), lambda i: (0, 0)),
        ],
        out_specs=pl.BlockSpec((_BE, D), lambda i: (i, 0)),
        out_shape=jax.ShapeDtypeStruct((E, D), jnp.float32),
    )(emb, ea, wfc1, wfc2)


_BN1 = 2000


def _node_lin_body(x0_ref, w_ref, out_ref):
    out_ref[...] = jnp.dot(x0_ref[...], w_ref[...], precision=_HI,
                           preferred_element_type=jnp.float32) * (1.0 / math.sqrt(D))


def _node_linear(x0, wlin1):
    grid = (N // _BN1,)
    return pl.pallas_call(
        _node_lin_body,
        grid=grid,
        in_specs=[
            pl.BlockSpec((_BN1, D), lambda i: (i, 0)),
            pl.BlockSpec((D, D), lambda i: (0, 0)),
        ],
        out_specs=pl.BlockSpec((_BN1, D), lambda i: (i, 0)),
        out_shape=jax.ShapeDtypeStruct((N, D), jnp.float32),
    )(x0, wlin1)


_NC = 2
_NS = 16
_NW = _NC * _NS
_CH = 128
_EPW = E // _NW
_NFULL = _EPW // _CH
_REM = _EPW - _NFULL * _CH
_RPT = 624
_RTAIL = N - _RPT * _NS

@functools.lru_cache(maxsize=1)
def _make_sc_kernel():
    mesh = plsc.VectorSubcoreMesh(core_axis_name="c", subcore_axis_name="s",
                                  num_cores=_NC, num_subcores=_NS)
    return pl.kernel(
        _sc_body,
        out_type=jax.ShapeDtypeStruct((_NC, N, D), jnp.float32),
        mesh=mesh,
        scratch_types=[
            pltpu.VMEM_SHARED((N, D), jnp.float32),
            pltpu.VMEM((_CH,), jnp.int32),
            pltpu.VMEM((_CH,), jnp.int32),
            pltpu.VMEM((_CH, D), jnp.float32),
            pltpu.VMEM((_CH, D), jnp.float32),
            pltpu.VMEM((_REM,), jnp.int32),
            pltpu.VMEM((_REM,), jnp.int32),
            pltpu.VMEM((_REM, D), jnp.float32),
            pltpu.VMEM((_REM, D), jnp.float32),
            pltpu.SemaphoreType.DMA,
        ],
    )


def _sc_body(x_hbm, wq_hbm, ei_hbm, ej_hbm, zero_hbm, out_hbm,
             acc, ei_v, ej_v, w_v, xg_v, ei_r, ej_r, w_r, xg_r, sem):
    cid = lax.axis_index("c")
    sid = lax.axis_index("s")
    pltpu.sync_copy(zero_hbm.at[pl.ds(sid * _RPT, _RPT)],
                    acc.at[pl.ds(sid * _RPT, _RPT)])

    @pl.when(sid == 0)
    def _zero_tail():
        pltpu.sync_copy(zero_hbm.at[pl.ds(_RPT * _NS, _RTAIL)],
                        acc.at[pl.ds(_RPT * _NS, _RTAIL)])
    plsc.subcore_barrier()

    wid = cid * _NS + sid
    ebase = wid * _EPW

    def chunk(k, carry):
        base = pl.multiple_of(ebase + k * _CH, 8)
        pltpu.sync_copy(ei_hbm.at[pl.ds(base, _CH)], ei_v)
        pltpu.sync_copy(ej_hbm.at[pl.ds(base, _CH)], ej_v)
        pltpu.sync_copy(wq_hbm.at[pl.ds(base, _CH)], w_v)
        pltpu.async_copy(x_hbm.at[ei_v], xg_v, sem).wait()

        def row(i, c2):
            for j in range(D // 16):
                sl = pl.ds(j * 16, 16)
                xg_v[i, sl] = xg_v[i, sl] * w_v[i, sl]
            return c2
        lax.fori_loop(0, _CH, row, 0)
        pltpu.sync_copy(xg_v, acc.at[ej_v], add=True)
        return carry

    lax.fori_loop(0, _NFULL, chunk, 0)

    rbase = pl.multiple_of(ebase + _NFULL * _CH, 8)
    pltpu.sync_copy(ei_hbm.at[pl.ds(rbase, _REM)], ei_r)
    pltpu.sync_copy(ej_hbm.at[pl.ds(rbase, _REM)], ej_r)
    pltpu.sync_copy(wq_hbm.at[pl.ds(rbase, _REM)], w_r)
    pltpu.async_copy(x_hbm.at[ei_r], xg_r, sem).wait()

    def rrow(i, c2):
        for j in range(D // 16):
            sl = pl.ds(j * 16, 16)
            xg_r[i, sl] = xg_r[i, sl] * w_r[i, sl]
        return c2
    lax.fori_loop(0, _REM, rrow, 0)
    pltpu.sync_copy(xg_r, acc.at[ej_r], add=True)

    plsc.subcore_barrier()
    pltpu.sync_copy(acc.at[pl.ds(sid * _RPT, _RPT)],
                    out_hbm.at[cid, pl.ds(sid * _RPT, _RPT)])

    @pl.when(sid == 0)
    def _dump_tail():
        pltpu.sync_copy(acc.at[pl.ds(_RPT * _NS, _RTAIL)],
                        out_hbm.at[cid, pl.ds(_RPT * _NS, _RTAIL)])


_BN3 = 400


def _final_body(agg2_ref, x0_ref, attrs_ref, wlin2_ref, wsc2d_ref, out_ref):
    agg = agg2_ref[0] + agg2_ref[1]
    z = jnp.dot(agg, wlin2_ref[...], precision=_HI,
                preferred_element_type=jnp.float32) * (1.0 / math.sqrt(D))
    x0 = x0_ref[...]
    t = jnp.dot(x0, wsc2d_ref[...], precision=_HI,
                preferred_element_type=jnp.float32)
    attrs = attrs_ref[...]
    sc = t[:, 0:D] * attrs[:, 0:1]
    for v in range(1, A):
        sc = sc + t[:, v * D:(v + 1) * D] * attrs[:, v:v + 1]
    sc = sc * (1.0 / math.sqrt(float(D * A)))
    out_ref[...] = _nsilu(z + sc) + x0


def _final(agg2, x0, attrs, wlin2, wsc2d):
    grid = (N // _BN3,)
    return pl.pallas_call(
        _final_body,
        grid=grid,
        in_specs=[
            pl.BlockSpec((_NC, _BN3, D), lambda i: (0, i, 0)),
            pl.BlockSpec((_BN3, D), lambda i: (i, 0)),
            pl.BlockSpec((_BN3, A), lambda i: (i, 0)),
            pl.BlockSpec((D, D), lambda i: (0, 0)),
            pl.BlockSpec((D, A * D), lambda i: (0, 0)),
        ],
        out_specs=pl.BlockSpec((_BN3, D), lambda i: (i, 0)),
        out_shape=jax.ShapeDtypeStruct((N, D), jnp.float32),
    )(agg2, x0, attrs, wlin2, wsc2d)


def kernel(node_features, node_attrs, edge_attrs, edge_embedding, edge_index,
           W_lin1, W_fc1, W_fc2, W_lin2, W_sc):
    ei = edge_index[0]
    ej = edge_index[1]
    x = _node_linear(node_features, W_lin1)
    wq = _edge_mlp(edge_embedding, edge_attrs, W_fc1, W_fc2)
    zeros = jnp.zeros((N, D), jnp.float32)
    agg2 = _make_sc_kernel()(x, wq, ei, ej, zeros)
    return _final(agg2, node_features, node_attrs, W_lin2, W_sc.reshape(D, A * D))
